# baseline (speedup 1.0000x reference)
import jax
import jax.numpy as jnp
from jax.experimental import pallas as pl
from jax.experimental.pallas import tpu as pltpu

_N_SEG = 8
_SEG = 1024
_D = 512
_H = 1024
_HSPLIT = 2
_HC = _H // _HSPLIT


def _ffn_body(x_ref, wg_ref, wu_ref, wd_ref, o_ref):
    j = pl.program_id(1)
    xb = x_ref[...].astype(jnp.bfloat16)
    wg = wg_ref[0].astype(jnp.bfloat16)
    wu = wu_ref[0].astype(jnp.bfloat16)
    g = jnp.dot(xb, wg, preferred_element_type=jnp.float32)
    u = jnp.dot(xb, wu, preferred_element_type=jnp.float32)
    h = ((g * jax.nn.sigmoid(g)) * u).astype(jnp.bfloat16)
    wd = wd_ref[0].astype(jnp.bfloat16)
    y = jnp.dot(h, wd, preferred_element_type=jnp.float32)

    @pl.when(j == 0)
    def _():
        o_ref[...] = y

    @pl.when(j != 0)
    def _():
        o_ref[...] += y


@jax.jit
def kernel(x, W_gate, W_up, W_down):
    grid = (_N_SEG, _HSPLIT)
    out = pl.pallas_call(
        _ffn_body,
        grid=grid,
        in_specs=[
            pl.BlockSpec((_SEG, _D), lambda i, j: (i, 0)),
            pl.BlockSpec((1, _D, _HC), lambda i, j: (i, 0, j)),
            pl.BlockSpec((1, _D, _HC), lambda i, j: (i, 0, j)),
            pl.BlockSpec((1, _HC, _D), lambda i, j: (i, j, 0)),
        ],
        out_specs=pl.BlockSpec((_SEG, _D), lambda i, j: (i, 0)),
        out_shape=jax.ShapeDtypeStruct((_N_SEG * _SEG, _D), jnp.float32),
        compiler_params=pltpu.CompilerParams(
            dimension_semantics=("arbitrary", "arbitrary"),
            vmem_limit_bytes=57 * 1024 * 1024,
        ),
    )(x, W_gate, W_up, W_down)
    return out


# speedup vs baseline: 1.1724x; 1.1724x over previous
import jax
import jax.numpy as jnp
from jax.experimental import pallas as pl
from jax.experimental.pallas import tpu as pltpu

_N_SEG = 8
_SEG = 1024
_D = 512
_H = 1024
_PER_STEP = 2


def _ffn_body(x_ref, wg_ref, wu_ref, wd_ref, o_ref):
    for k in range(_PER_STEP):
        xb = x_ref[k * _SEG:(k + 1) * _SEG, :].astype(jnp.bfloat16)
        wg = wg_ref[k].astype(jnp.bfloat16)
        wu = wu_ref[k].astype(jnp.bfloat16)
        g = jnp.dot(xb, wg, preferred_element_type=jnp.float32)
        u = jnp.dot(xb, wu, preferred_element_type=jnp.float32)
        h = ((g * jax.nn.sigmoid(g)) * u).astype(jnp.bfloat16)
        wd = wd_ref[k].astype(jnp.bfloat16)
        o_ref[k * _SEG:(k + 1) * _SEG, :] = jnp.dot(
            h, wd, preferred_element_type=jnp.float32)


@jax.jit
def kernel(x, W_gate, W_up, W_down):
    grid = (_N_SEG // _PER_STEP,)
    out = pl.pallas_call(
        _ffn_body,
        grid=grid,
        in_specs=[
            pl.BlockSpec((_PER_STEP * _SEG, _D), lambda i: (i, 0)),
            pl.BlockSpec((_PER_STEP, _D, _H), lambda i: (i, 0, 0)),
            pl.BlockSpec((_PER_STEP, _D, _H), lambda i: (i, 0, 0)),
            pl.BlockSpec((_PER_STEP, _H, _D), lambda i: (i, 0, 0)),
        ],
        out_specs=pl.BlockSpec((_PER_STEP * _SEG, _D), lambda i: (i, 0)),
        out_shape=jax.ShapeDtypeStruct((_N_SEG * _SEG, _D), jnp.float32),
        compiler_params=pltpu.CompilerParams(
            dimension_semantics=("arbitrary",),
            vmem_limit_bytes=57 * 1024 * 1024,
        ),
    )(x, W_gate, W_up, W_down)
    return out
